# baseline (speedup 1.0000x reference)
import jax
import jax.numpy as jnp
from jax import lax
from jax.experimental import pallas as pl
from jax.experimental.pallas import tpu as pltpu
from jax.experimental.pallas import tpu_sc as plsc

VOCAB = 100000
D = 64
B = 4096
CTX = 20
NC = 2
NS = 16
NW = NC * NS
BPW = B // NW
LANES = 16


PASSES = 8
PBW = BPW // PASSES


def _sc_pool_body(idx_hbm, table_hbm, out_hbm, idx_v, rows_v, out_v, sems):
    wid = lax.axis_index("s") * NC + lax.axis_index("c")
    base = wid * BPW
    pltpu.sync_copy(idx_hbm.at[:, pl.ds(base, BPW)], idx_v)

    def fire(p):
        b = p % 2
        ds_ = []
        for j in range(CTX):
            d = pltpu.make_async_copy(
                table_hbm.at[idx_v.at[j, pl.ds(p * PBW, PBW)]],
                rows_v.at[b, j],
                sems.at[b],
            )
            d.start()
            ds_.append(d)
        return ds_

    pend = fire(0)
    for p in range(PASSES):
        for d in pend:
            d.wait()
        if p + 1 < PASSES:
            pend = fire(p + 1)
        b = p % 2

        def body(r, carry):
            for q in range(D // LANES):
                sl = pl.ds(q * LANES, LANES)
                acc = rows_v[b, 0, r, sl]
                for j in range(1, CTX):
                    acc = acc + rows_v[b, j, r, sl]
                out_v[p * PBW + r, sl] = acc * (1.0 / CTX)
            return carry

        lax.fori_loop(0, PBW, body, 0)

    pltpu.sync_copy(out_v, out_hbm.at[pl.ds(base, BPW)])


_sc_pool = pl.kernel(
    _sc_pool_body,
    out_type=jax.ShapeDtypeStruct((B, D), jnp.float32),
    mesh=plsc.VectorSubcoreMesh(core_axis_name="c", subcore_axis_name="s"),
    scratch_types=[
        pltpu.VMEM((CTX, BPW), jnp.int32),
        pltpu.VMEM((2, CTX, PBW, 2 * D), jnp.float32),
        pltpu.VMEM((BPW, D), jnp.float32),
        pltpu.SemaphoreType.DMA((2,)),
    ],
    compiler_params=pltpu.CompilerParams(use_tc_tiling_on_sc=False),
)

VB = 1024


def _mm_body(w_ref, p_ref, o_ref):
    o_ref[...] = lax.dot_general(
        w_ref[...], p_ref[...],
        dimension_numbers=(((0,), (1,)), ((), ())),
        preferred_element_type=jnp.float32,
    )


def _matmul_t(wt, pooled):
    return pl.pallas_call(
        _mm_body,
        grid=(pl.cdiv(VOCAB, VB),),
        in_specs=[
            pl.BlockSpec((D, VB), lambda j: (0, j)),
            pl.BlockSpec((B, D), lambda j: (0, 0)),
        ],
        out_specs=pl.BlockSpec((VB, B), lambda j: (j, 0)),
        out_shape=jax.ShapeDtypeStruct((VOCAB, B), jnp.float32),
    )(wt, pooled)


@jax.jit
def kernel(context_words, emb_table, linear_w):
    idx = context_words.astype(jnp.int32).T
    wt = linear_w.T
    table = jnp.pad(emb_table, ((0, 0), (0, D)))
    pooled = _sc_pool(idx, table)
    return _matmul_t(wt, pooled).T


# speedup vs baseline: 1.0055x; 1.0055x over previous
import jax
import jax.numpy as jnp
from jax import lax
from jax.experimental import pallas as pl
from jax.experimental.pallas import tpu as pltpu
from jax.experimental.pallas import tpu_sc as plsc

VOCAB = 100000
D = 64
B = 4096
CTX = 20
NC = 2
NS = 16
NW = NC * NS
BPW = B // NW
LANES = 16


PASSES = 4
PBW = BPW // PASSES


def _sc_pool_body(idx_hbm, table_hbm, out_hbm, idx_v, rows_v, out_v, sems):
    wid = lax.axis_index("s") * NC + lax.axis_index("c")
    base = wid * BPW
    pltpu.sync_copy(idx_hbm.at[:, pl.ds(base, BPW)], idx_v)

    def fire(p):
        b = p % 2
        ds_ = []
        for j in range(CTX):
            d = pltpu.make_async_copy(
                table_hbm.at[idx_v.at[j, pl.ds(p * PBW, PBW)]],
                rows_v.at[b, j],
                sems.at[b],
            )
            d.start()
            ds_.append(d)
        return ds_

    pend = fire(0)
    for p in range(PASSES):
        for d in pend:
            d.wait()
        if p + 1 < PASSES:
            pend = fire(p + 1)
        b = p % 2

        def body(r, carry):
            for q in range(D // LANES):
                sl = pl.ds(q * LANES, LANES)
                acc = rows_v[b, 0, r, sl]
                for j in range(1, CTX):
                    acc = acc + rows_v[b, j, r, sl]
                out_v[p * PBW + r, sl] = acc * (1.0 / CTX)
            return carry

        lax.fori_loop(0, PBW, body, 0)

    pltpu.sync_copy(out_v, out_hbm.at[pl.ds(base, BPW)])


_sc_pool = pl.kernel(
    _sc_pool_body,
    out_type=jax.ShapeDtypeStruct((B, D), jnp.float32),
    mesh=plsc.VectorSubcoreMesh(core_axis_name="c", subcore_axis_name="s"),
    scratch_types=[
        pltpu.VMEM((CTX, BPW), jnp.int32),
        pltpu.VMEM((2, CTX, PBW, D), jnp.float32),
        pltpu.VMEM((BPW, D), jnp.float32),
        pltpu.SemaphoreType.DMA((2,)),
    ],
    compiler_params=pltpu.CompilerParams(use_tc_tiling_on_sc=False),
)

VB = 1024


def _mm_body(w_ref, p_ref, o_ref):
    o_ref[...] = lax.dot_general(
        w_ref[...], p_ref[...],
        dimension_numbers=(((0,), (1,)), ((), ())),
        preferred_element_type=jnp.float32,
    )


def _matmul_t(wt, pooled):
    return pl.pallas_call(
        _mm_body,
        grid=(pl.cdiv(VOCAB, VB),),
        in_specs=[
            pl.BlockSpec((D, VB), lambda j: (0, j)),
            pl.BlockSpec((B, D), lambda j: (0, 0)),
        ],
        out_specs=pl.BlockSpec((VB, B), lambda j: (j, 0)),
        out_shape=jax.ShapeDtypeStruct((VOCAB, B), jnp.float32),
    )(wt, pooled)


@jax.jit
def kernel(context_words, emb_table, linear_w):
    idx = context_words.astype(jnp.int32).T
    wt = linear_w.T
    pooled = _sc_pool(idx, emb_table)
    return _matmul_t(wt, pooled).T
